# baseline (speedup 1.0000x reference)
import jax
import jax.numpy as jnp
from jax import lax
from jax.experimental import pallas as pl
from jax.experimental.pallas import tpu as pltpu
from jax.experimental.pallas import tpu_sc as plsc

B = 16384
D = 64
V = 100000
NC = 2
NS = 16
NW = NC * NS
RPW = 2
IC = 2048


def _sc_body(uidx_hbm, iidx_hbm, ug_hbm, ig_hbm, um_hbm, im_hbm,
             ug_out, ig_out, um_out, im_out,
             rowbuf, idxbuf, ob0, ob1, sem, osem):
  wid = lax.axis_index("s") * NC + lax.axis_index("c")
  zero16 = jnp.zeros((16,), jnp.int32)
  obufs = (ob0, ob1)
  pending = [None, None]

  def gather_rows(tab_hbm, out_hbm):
    for r in range(RPW):
      j = wid * RPW + r
      pltpu.sync_copy(tab_hbm.at[pl.ds(j, 1), :], rowbuf)
      for c in range(B // IC):
        ob = obufs[c % 2]
        if pending[c % 2] is not None:
          pending[c % 2].wait()

        def vec_body(v, _, c=c, ob=ob):
          for u in range(16):
            off = v * 256 + u * 16
            iv = idxbuf[pl.ds(c * IC + off, 16)]
            g = plsc.load_gather(rowbuf, [zero16, iv])
            ob[0, pl.ds(off, 16)] = g
          return _

        lax.fori_loop(0, IC // 256, vec_body, 0)
        cp = pltpu.async_copy(
            ob, out_hbm.at[pl.ds(j, 1), pl.ds(c * IC, IC)], osem)
        pending[c % 2] = cp

  pltpu.sync_copy(uidx_hbm, idxbuf)
  gather_rows(ug_hbm, ug_out)
  gather_rows(um_hbm, um_out)
  pltpu.sync_copy(iidx_hbm, idxbuf)
  gather_rows(ig_hbm, ig_out)
  gather_rows(im_hbm, im_out)
  for p in pending:
    if p is not None:
      p.wait()


_sc_gather = pl.kernel(
    _sc_body,
    out_type=[jax.ShapeDtypeStruct((D, B), jnp.float32)] * 4,
    mesh=plsc.VectorSubcoreMesh(core_axis_name="c", subcore_axis_name="s"),
    scratch_types=[
        pltpu.VMEM((1, V), jnp.float32),
        pltpu.VMEM((B,), jnp.int32),
        pltpu.VMEM((1, IC), jnp.float32),
        pltpu.VMEM((1, IC), jnp.float32),
        pltpu.SemaphoreType.DMA,
        pltpu.SemaphoreType.DMA,
    ],
    compiler_params=pltpu.CompilerParams(needs_layout_passes=False),
)

BLKC = 2048


def _dense_body(ug_ref, ig_ref, um_ref, im_ref, w1_ref, b1_ref, wp_ref,
                bp_ref, out_ref):
  gmf = ug_ref[...] * ig_ref[...]
  h = jnp.dot(w1_ref[:, :D], um_ref[...], preferred_element_type=jnp.float32)
  h = h + jnp.dot(w1_ref[:, D:], im_ref[...],
                  preferred_element_type=jnp.float32)
  h = jnp.maximum(h + b1_ref[...], 0.0)
  pred = jnp.dot(wp_ref[:, :D], gmf, preferred_element_type=jnp.float32)
  pred = pred + jnp.dot(wp_ref[:, D:], h, preferred_element_type=jnp.float32)
  out_ref[...] = pred[0, :] + bp_ref[0, 0]


def _dense_call(ug_t, ig_t, um_t, im_t, W1, b1_2d, wp, bp_2d):
  grid = (B // BLKC,)
  col_spec = pl.BlockSpec((D, BLKC), lambda i: (0, i))
  return pl.pallas_call(
      _dense_body,
      grid=grid,
      in_specs=[
          col_spec, col_spec, col_spec, col_spec,
          pl.BlockSpec((D, 2 * D), lambda i: (0, 0)),
          pl.BlockSpec((D, 1), lambda i: (0, 0)),
          pl.BlockSpec((1, 2 * D), lambda i: (0, 0)),
          pl.BlockSpec((1, 1), lambda i: (0, 0)),
      ],
      out_specs=pl.BlockSpec((BLKC,), lambda i: (i,)),
      out_shape=jax.ShapeDtypeStruct((B,), jnp.float32),
  )(ug_t, ig_t, um_t, im_t, W1, b1_2d, wp, bp_2d)


def kernel(user_indices, item_indices, user_gmf_table, item_gmf_table,
           user_mlp_table, item_mlp_table, W1, b1, Wp, bp):
  uidx = user_indices.astype(jnp.int32)
  iidx = item_indices.astype(jnp.int32)
  ug_t, ig_t, um_t, im_t = _sc_gather(
      uidx, iidx, user_gmf_table.T, item_gmf_table.T,
      user_mlp_table.T, item_mlp_table.T)
  return _dense_call(ug_t, ig_t, um_t, im_t, W1, b1.reshape(D, 1), Wp,
                     bp.reshape(1, 1))


# speedup vs baseline: 1.0660x; 1.0660x over previous
import jax
import jax.numpy as jnp
from jax import lax
from jax.experimental import pallas as pl
from jax.experimental.pallas import tpu as pltpu
from jax.experimental.pallas import tpu_sc as plsc

B = 16384
D = 64
V = 100000
NC = 2
NS = 16
NW = NC * NS
RPW = 2
IC = 4096


def _sc_body(uidx_hbm, iidx_hbm, ug_hbm, ig_hbm, um_hbm, im_hbm,
             ug_out, ig_out, um_out, im_out,
             rowbuf, idxbuf, ob0, ob1, sem, osem):
  wid = lax.axis_index("s") * NC + lax.axis_index("c")
  zero16 = jnp.zeros((16,), jnp.int32)
  obufs = (ob0, ob1)
  pending = [None, None]

  def gather_rows(tab_hbm, out_hbm):
    for r in range(RPW):
      j = wid * RPW + r
      pltpu.sync_copy(tab_hbm.at[pl.ds(j, 1), :], rowbuf)
      for c in range(B // IC):
        ob = obufs[c % 2]
        if pending[c % 2] is not None:
          pending[c % 2].wait()

        def vec_body(v, _, c=c, ob=ob):
          for u in range(8):
            off = v * 128 + u * 16
            iv = idxbuf[pl.ds(c * IC + off, 16)]
            g = plsc.load_gather(rowbuf, [zero16, iv])
            ob[0, pl.ds(off, 16)] = g
          return _

        lax.fori_loop(0, IC // 128, vec_body, 0)
        cp = pltpu.async_copy(
            ob, out_hbm.at[pl.ds(j, 1), pl.ds(c * IC, IC)], osem)
        pending[c % 2] = cp

  pltpu.sync_copy(uidx_hbm, idxbuf)
  gather_rows(ug_hbm, ug_out)
  gather_rows(um_hbm, um_out)
  pltpu.sync_copy(iidx_hbm, idxbuf)
  gather_rows(ig_hbm, ig_out)
  gather_rows(im_hbm, im_out)
  for p in pending:
    if p is not None:
      p.wait()


_sc_gather = pl.kernel(
    _sc_body,
    out_type=[jax.ShapeDtypeStruct((D, B), jnp.float32)] * 4,
    mesh=plsc.VectorSubcoreMesh(core_axis_name="c", subcore_axis_name="s"),
    scratch_types=[
        pltpu.VMEM((1, V), jnp.float32),
        pltpu.VMEM((B,), jnp.int32),
        pltpu.VMEM((1, IC), jnp.float32),
        pltpu.VMEM((1, IC), jnp.float32),
        pltpu.SemaphoreType.DMA,
        pltpu.SemaphoreType.DMA,
    ],
    compiler_params=pltpu.CompilerParams(needs_layout_passes=False),
)

BLKC = 2048


def _dense_body(ug_ref, ig_ref, um_ref, im_ref, w1_ref, b1_ref, wp_ref,
                bp_ref, out_ref):
  gmf = ug_ref[...] * ig_ref[...]
  h = jnp.dot(w1_ref[:, :D], um_ref[...], preferred_element_type=jnp.float32)
  h = h + jnp.dot(w1_ref[:, D:], im_ref[...],
                  preferred_element_type=jnp.float32)
  h = jnp.maximum(h + b1_ref[...], 0.0)
  pred = jnp.dot(wp_ref[:, :D], gmf, preferred_element_type=jnp.float32)
  pred = pred + jnp.dot(wp_ref[:, D:], h, preferred_element_type=jnp.float32)
  out_ref[...] = pred[0, :] + bp_ref[0, 0]


def _dense_call(ug_t, ig_t, um_t, im_t, W1, b1_2d, wp, bp_2d):
  grid = (B // BLKC,)
  col_spec = pl.BlockSpec((D, BLKC), lambda i: (0, i))
  return pl.pallas_call(
      _dense_body,
      grid=grid,
      in_specs=[
          col_spec, col_spec, col_spec, col_spec,
          pl.BlockSpec((D, 2 * D), lambda i: (0, 0)),
          pl.BlockSpec((D, 1), lambda i: (0, 0)),
          pl.BlockSpec((1, 2 * D), lambda i: (0, 0)),
          pl.BlockSpec((1, 1), lambda i: (0, 0)),
      ],
      out_specs=pl.BlockSpec((BLKC,), lambda i: (i,)),
      out_shape=jax.ShapeDtypeStruct((B,), jnp.float32),
  )(ug_t, ig_t, um_t, im_t, W1, b1_2d, wp, bp_2d)


def kernel(user_indices, item_indices, user_gmf_table, item_gmf_table,
           user_mlp_table, item_mlp_table, W1, b1, Wp, bp):
  uidx = user_indices.astype(jnp.int32)
  iidx = item_indices.astype(jnp.int32)
  ug_t, ig_t, um_t, im_t = _sc_gather(
      uidx, iidx, user_gmf_table.T, item_gmf_table.T,
      user_mlp_table.T, item_mlp_table.T)
  return _dense_call(ug_t, ig_t, um_t, im_t, W1, b1.reshape(D, 1), Wp,
                     bp.reshape(1, 1))


# speedup vs baseline: 1.0873x; 1.0199x over previous
import jax
import jax.numpy as jnp
from jax import lax
from jax.experimental import pallas as pl
from jax.experimental.pallas import tpu as pltpu
from jax.experimental.pallas import tpu_sc as plsc

B = 16384
D = 64
V = 100000
NC = 2
NS = 16
NW = NC * NS
RPW = 2
IC = 4096


def _sc_body(uidx_hbm, iidx_hbm, ug_hbm, ig_hbm, um_hbm, im_hbm,
             ug_out, ig_out, um_out, im_out,
             rowbuf, idxbuf, ob0, ob1, sem, osem):
  wid = lax.axis_index("s") * NC + lax.axis_index("c")
  zero16 = jnp.zeros((16,), jnp.int32)
  obufs = (ob0, ob1)
  pending = [None, None]

  def gather_rows(tab_hbm, out_hbm):
    for r in range(RPW):
      j = wid * RPW + r
      pltpu.sync_copy(tab_hbm.at[pl.ds(j, 1), :], rowbuf)
      for c in range(B // IC):
        ob = obufs[c % 2]
        if pending[c % 2] is not None:
          pending[c % 2].wait()

        def vec_body(v, _, c=c, ob=ob):
          for u in range(8):
            off = v * 128 + u * 16
            iv = idxbuf[pl.ds(c * IC + off, 16)]
            g = plsc.load_gather(rowbuf, [zero16, iv])
            ob[0, pl.ds(off, 16)] = g
          return _

        lax.fori_loop(0, IC // 128, vec_body, 0)
        cp = pltpu.async_copy(
            ob, out_hbm.at[pl.ds(j, 1), pl.ds(c * IC, IC)], osem)
        pending[c % 2] = cp

  pltpu.sync_copy(uidx_hbm, idxbuf)
  gather_rows(ug_hbm, ug_out)
  gather_rows(um_hbm, um_out)
  pltpu.sync_copy(iidx_hbm, idxbuf)
  gather_rows(ig_hbm, ig_out)
  gather_rows(im_hbm, im_out)
  for p in pending:
    if p is not None:
      p.wait()


_sc_gather = pl.kernel(
    _sc_body,
    out_type=[jax.ShapeDtypeStruct((D, B), jnp.float32)] * 4,
    mesh=plsc.VectorSubcoreMesh(core_axis_name="c", subcore_axis_name="s"),
    scratch_types=[
        pltpu.VMEM((1, V), jnp.float32),
        pltpu.VMEM((B,), jnp.int32),
        pltpu.VMEM((1, IC), jnp.float32),
        pltpu.VMEM((1, IC), jnp.float32),
        pltpu.SemaphoreType.DMA,
        pltpu.SemaphoreType.DMA,
    ],
    compiler_params=pltpu.CompilerParams(needs_layout_passes=False),
)

BLKC = 4096


def _dense_body(ug_ref, ig_ref, um_ref, im_ref, w1_ref, b1_ref, wp_ref,
                bp_ref, out_ref):
  gmf = ug_ref[...] * ig_ref[...]
  h = jnp.dot(w1_ref[:, :D], um_ref[...], preferred_element_type=jnp.float32)
  h = h + jnp.dot(w1_ref[:, D:], im_ref[...],
                  preferred_element_type=jnp.float32)
  h = jnp.maximum(h + b1_ref[...], 0.0)
  pred = jnp.dot(wp_ref[:, :D], gmf, preferred_element_type=jnp.float32)
  pred = pred + jnp.dot(wp_ref[:, D:], h, preferred_element_type=jnp.float32)
  out_ref[...] = pred[0, :] + bp_ref[0, 0]


def _dense_call(ug_t, ig_t, um_t, im_t, W1, b1_2d, wp, bp_2d):
  grid = (B // BLKC,)
  col_spec = pl.BlockSpec((D, BLKC), lambda i: (0, i))
  return pl.pallas_call(
      _dense_body,
      grid=grid,
      in_specs=[
          col_spec, col_spec, col_spec, col_spec,
          pl.BlockSpec((D, 2 * D), lambda i: (0, 0)),
          pl.BlockSpec((D, 1), lambda i: (0, 0)),
          pl.BlockSpec((1, 2 * D), lambda i: (0, 0)),
          pl.BlockSpec((1, 1), lambda i: (0, 0)),
      ],
      out_specs=pl.BlockSpec((BLKC,), lambda i: (i,)),
      out_shape=jax.ShapeDtypeStruct((B,), jnp.float32),
  )(ug_t, ig_t, um_t, im_t, W1, b1_2d, wp, bp_2d)


def kernel(user_indices, item_indices, user_gmf_table, item_gmf_table,
           user_mlp_table, item_mlp_table, W1, b1, Wp, bp):
  uidx = user_indices.astype(jnp.int32)
  iidx = item_indices.astype(jnp.int32)
  ug_t, ig_t, um_t, im_t = _sc_gather(
      uidx, iidx, user_gmf_table.T, item_gmf_table.T,
      user_mlp_table.T, item_mlp_table.T)
  return _dense_call(ug_t, ig_t, um_t, im_t, W1, b1.reshape(D, 1), Wp,
                     bp.reshape(1, 1))


# speedup vs baseline: 1.0943x; 1.0065x over previous
import jax
import jax.numpy as jnp
from jax import lax
from jax.experimental import pallas as pl
from jax.experimental.pallas import tpu as pltpu
from jax.experimental.pallas import tpu_sc as plsc

B = 16384
D = 64
V = 100000
NC = 2
NS = 16
NW = NC * NS
RPW = 2
IC = 4096


def _sc_body(uidx_hbm, iidx_hbm, ug_hbm, ig_hbm, um_hbm, im_hbm,
             ug_out, ig_out, um_out, im_out,
             rowbuf, idxbuf, ob0, ob1, sem, osem):
  wid = lax.axis_index("s") * NC + lax.axis_index("c")
  zero16 = jnp.zeros((16,), jnp.int32)
  obufs = (ob0, ob1)
  pending = [None, None]

  def gather_rows(tab_hbm, out_hbm):
    for r in range(RPW):
      j = wid * RPW + r
      pltpu.sync_copy(tab_hbm.at[pl.ds(j, 1), :], rowbuf)
      for c in range(B // IC):
        ob = obufs[c % 2]
        if pending[c % 2] is not None:
          pending[c % 2].wait()

        def vec_body(v, _, c=c, ob=ob):
          for u in range(8):
            off = v * 128 + u * 16
            iv = idxbuf[pl.ds(c * IC + off, 16)]
            g = plsc.load_gather(rowbuf, [zero16, iv])
            ob[0, pl.ds(off, 16)] = g
          return _

        lax.fori_loop(0, IC // 128, vec_body, 0)
        cp = pltpu.async_copy(
            ob, out_hbm.at[pl.ds(j, 1), pl.ds(c * IC, IC)], osem)
        pending[c % 2] = cp

  pltpu.sync_copy(uidx_hbm, idxbuf)
  gather_rows(ug_hbm, ug_out)
  gather_rows(um_hbm, um_out)
  pltpu.sync_copy(iidx_hbm, idxbuf)
  gather_rows(ig_hbm, ig_out)
  gather_rows(im_hbm, im_out)
  for p in pending:
    if p is not None:
      p.wait()


_sc_gather = pl.kernel(
    _sc_body,
    out_type=[jax.ShapeDtypeStruct((D, B), jnp.float32)] * 4,
    mesh=plsc.VectorSubcoreMesh(core_axis_name="c", subcore_axis_name="s"),
    scratch_types=[
        pltpu.VMEM((1, V), jnp.float32),
        pltpu.VMEM((B,), jnp.int32),
        pltpu.VMEM((1, IC), jnp.float32),
        pltpu.VMEM((1, IC), jnp.float32),
        pltpu.SemaphoreType.DMA,
        pltpu.SemaphoreType.DMA,
    ],
    compiler_params=pltpu.CompilerParams(needs_layout_passes=False),
)

BLKC = 8192


def _dense_body(ug_ref, ig_ref, um_ref, im_ref, w1_ref, b1_ref, wp_ref,
                bp_ref, out_ref):
  gmf = ug_ref[...] * ig_ref[...]
  h = jnp.dot(w1_ref[:, :D], um_ref[...], preferred_element_type=jnp.float32)
  h = h + jnp.dot(w1_ref[:, D:], im_ref[...],
                  preferred_element_type=jnp.float32)
  h = jnp.maximum(h + b1_ref[...], 0.0)
  pred = jnp.dot(wp_ref[:, :D], gmf, preferred_element_type=jnp.float32)
  pred = pred + jnp.dot(wp_ref[:, D:], h, preferred_element_type=jnp.float32)
  out_ref[...] = pred[0, :] + bp_ref[0, 0]


def _dense_call(ug_t, ig_t, um_t, im_t, W1, b1_2d, wp, bp_2d):
  grid = (B // BLKC,)
  col_spec = pl.BlockSpec((D, BLKC), lambda i: (0, i))
  return pl.pallas_call(
      _dense_body,
      grid=grid,
      in_specs=[
          col_spec, col_spec, col_spec, col_spec,
          pl.BlockSpec((D, 2 * D), lambda i: (0, 0)),
          pl.BlockSpec((D, 1), lambda i: (0, 0)),
          pl.BlockSpec((1, 2 * D), lambda i: (0, 0)),
          pl.BlockSpec((1, 1), lambda i: (0, 0)),
      ],
      out_specs=pl.BlockSpec((BLKC,), lambda i: (i,)),
      out_shape=jax.ShapeDtypeStruct((B,), jnp.float32),
  )(ug_t, ig_t, um_t, im_t, W1, b1_2d, wp, bp_2d)


def kernel(user_indices, item_indices, user_gmf_table, item_gmf_table,
           user_mlp_table, item_mlp_table, W1, b1, Wp, bp):
  uidx = user_indices.astype(jnp.int32)
  iidx = item_indices.astype(jnp.int32)
  ug_t, ig_t, um_t, im_t = _sc_gather(
      uidx, iidx, user_gmf_table.T, item_gmf_table.T,
      user_mlp_table.T, item_mlp_table.T)
  return _dense_call(ug_t, ig_t, um_t, im_t, W1, b1.reshape(D, 1), Wp,
                     bp.reshape(1, 1))
